# baseline (speedup 1.0000x reference)
import functools

import jax
import jax.numpy as jnp
from jax import lax
from jax.experimental import pallas as pl
from jax.experimental.pallas import tpu as pltpu
from jax.experimental.pallas import tpu_sc as plsc

B, L, T = 64, 1632, 408
H, D, K = 64, 512, 1024
BT = B * T
TOK = 768
NTILE = BT // TOK
NC, NS = 2, 16
NW = NC * NS
BPW = BT // NW
CH = 48
NCHUNK = BPW // CH

_HI = lax.Precision.HIGHEST


def _dot(a, b, precision=_HI):
    return jnp.dot(a, b, preferred_element_type=jnp.float32, precision=precision)


def _encode(img_blk, w14, b14, w2c, b2, w3, b3):
    h1 = jnp.maximum(_dot(img_blk, w14) + b14, 0.0)
    h2 = jnp.maximum(_dot(h1, w2c) + b2, 0.0)
    return _dot(h2, w3) + b3


def _enc_vq_body(img_ref, w14_ref, b14_ref, w2_ref, b2_ref, w3_ref, b3_ref,
                 cbt_ref, idx_ref, cbt_bf, c2_ref):
    @pl.when(pl.program_id(0) == 0)
    def _init():
        cbt = cbt_ref[...]
        cbt_bf[...] = cbt.astype(jnp.bfloat16)
        c2_ref[...] = jnp.sum(cbt * cbt, axis=0, keepdims=True)

    x = _encode(img_ref[...], w14_ref[...], b14_ref[...], w2_ref[...],
                b2_ref[...], w3_ref[...], b3_ref[...])
    e = _dot(x.astype(jnp.bfloat16), cbt_bf[...], precision=None)
    x2 = jnp.sum(x * x, axis=1, keepdims=True)
    dist = (x2 - 2.0 * e) + c2_ref[...]
    m = jnp.min(dist, axis=1, keepdims=True)
    iota = lax.broadcasted_iota(jnp.int32, dist.shape, 1)
    idx_ref[...] = jnp.min(jnp.where(dist == m, iota, K), axis=1)[None, None]


def _vq_indices(imgr, w14, b14, w2c, b2, w3, b3, cbt):
    rep = lambda i: (0, 0)
    return pl.pallas_call(
        _enc_vq_body,
        grid=(NTILE,),
        in_specs=[
            pl.BlockSpec((TOK, 4), lambda i: (i, 0)),
            pl.BlockSpec((4, 128), rep),
            pl.BlockSpec((1, 128), rep),
            pl.BlockSpec((128, H), rep),
            pl.BlockSpec((1, H), rep),
            pl.BlockSpec((H, D), rep),
            pl.BlockSpec((1, D), rep),
            pl.BlockSpec((D, K), rep),
        ],
        out_specs=pl.BlockSpec((1, 1, TOK), lambda i: (i, 0, 0)),
        out_shape=jax.ShapeDtypeStruct((NTILE, 1, TOK), jnp.int32),
        scratch_shapes=[
            pltpu.VMEM((D, K), jnp.bfloat16),
            pltpu.VMEM((1, K), jnp.float32),
        ],
    )(imgr, w14, b14, w2c, b2, w3, b3, cbt)


def _sc_gather_body(cb_hbm, idx_hbm, out_hbm, idx_v, rows0, rows1, sem0, sem1):
    wid = lax.axis_index("s") * NC + lax.axis_index("c")
    base = wid * BPW
    pltpu.sync_copy(idx_hbm.at[wid], idx_v)
    rows = (rows0, rows1)
    sems = (sem0, sem1)
    pending = pltpu.async_copy(cb_hbm.at[idx_v.at[0]], rows0, sem0)
    for j in range(NCHUNK):
        nxt = None
        if j + 1 < NCHUNK:
            nxt = pltpu.async_copy(cb_hbm.at[idx_v.at[j + 1]],
                                   rows[(j + 1) % 2], sems[(j + 1) % 2])
        pending.wait()
        pltpu.sync_copy(rows[j % 2], out_hbm.at[pl.ds(base + j * CH, CH)])
        pending = nxt


def _sc_gather(codebook, idx3):
    return pl.kernel(
        _sc_gather_body,
        mesh=plsc.VectorSubcoreMesh(core_axis_name="c", subcore_axis_name="s",
                                    num_cores=NC, num_subcores=NS),
        out_type=jax.ShapeDtypeStruct((BT, D), jnp.float32),
        scratch_types=[
            pltpu.VMEM((NCHUNK, CH), jnp.int32),
            pltpu.VMEM((CH, D), jnp.float32),
            pltpu.VMEM((CH, D), jnp.float32),
            pltpu.SemaphoreType.DMA,
            pltpu.SemaphoreType.DMA,
        ],
    )(codebook, idx3)


def _dec_body(img_ref, q_ref, w14_ref, b14_ref, w2_ref, b2_ref, w3_ref, b3_ref,
              vd0_ref, vd1_ref, bd1_ref, w2cat_ref, bd2_ref, wsel_ref, bd3_ref,
              out_ref, quant_ref, recon_ref, commit_ref):
    b = pl.program_id(0)
    img_blk = img_ref[0]
    qb = q_ref[...]
    x = _encode(img_blk, w14_ref[...], b14_ref[...], w2_ref[...],
                b2_ref[...], w3_ref[...], b3_ref[...])
    a0 = jnp.maximum(_dot(qb, vd0_ref[...]) + bd1_ref[...], 0.0)
    a1 = jnp.maximum(_dot(qb, vd1_ref[...]) + bd1_ref[...], 0.0)
    f0 = jnp.maximum(_dot(a0, w2cat_ref[...]) + bd2_ref[...], 0.0)
    f1 = jnp.maximum(_dot(a1, w2cat_ref[...]) + bd2_ref[...], 0.0)
    o01 = _dot(f0, wsel_ref[...])
    o23 = _dot(f1, wsel_ref[...])
    out_blk = jnp.concatenate([o01, o23], axis=1) + bd3_ref[0, 0]
    out_ref[...] = out_blk[None]
    quant_ref[...] = jnp.transpose(qb)[None]

    s_r = jnp.sum((img_blk - out_blk) ** 2)
    s_c = jnp.sum((qb - x) ** 2)
    pr = jnp.where(b == 0, 0.0, recon_ref[0, 0])
    pc = jnp.where(b == 0, 0.0, commit_ref[0, 0])
    last = b == B - 1
    recon_ref[0, 0] = jnp.where(last, (pr + s_r) / (B * L), pr + s_r)
    commit_ref[0, 0] = jnp.where(last, (pc + s_c) / (BT * D), pc + s_c)


def _decode(imgr4, q, w14, b14, w2c, b2, w3, b3, vd0, vd1, bd1, w2cat, bd2,
            wsel, bd3):
    rep = lambda b: (0, 0)
    return pl.pallas_call(
        _dec_body,
        grid=(B,),
        in_specs=[
            pl.BlockSpec((1, T, 4), lambda b: (b, 0, 0)),
            pl.BlockSpec((T, D), lambda b: (b, 0)),
            pl.BlockSpec((4, 128), rep),
            pl.BlockSpec((1, 128), rep),
            pl.BlockSpec((128, H), rep),
            pl.BlockSpec((1, H), rep),
            pl.BlockSpec((H, D), rep),
            pl.BlockSpec((1, D), rep),
            pl.BlockSpec((D, H), rep),
            pl.BlockSpec((D, H), rep),
            pl.BlockSpec((1, H), rep),
            pl.BlockSpec((H, 128), rep),
            pl.BlockSpec((1, 128), rep),
            pl.BlockSpec((128, 2), rep),
            pl.BlockSpec((1, 1), rep),
        ],
        out_specs=[
            pl.BlockSpec((1, T, 4), lambda b: (b, 0, 0)),
            pl.BlockSpec((1, D, T), lambda b: (b, 0, 0)),
            pl.BlockSpec(memory_space=pltpu.SMEM),
            pl.BlockSpec(memory_space=pltpu.SMEM),
        ],
        out_shape=[
            jax.ShapeDtypeStruct((B, T, 4), jnp.float32),
            jax.ShapeDtypeStruct((B, D, T), jnp.float32),
            jax.ShapeDtypeStruct((1, 1), jnp.float32),
            jax.ShapeDtypeStruct((1, 1), jnp.float32),
        ],
    )(imgr4, q, w14, b14, w2c, b2, w3, b3, vd0, vd1, bd1, w2cat, bd2, wsel,
      bd3)


def kernel(img, enc_w1, enc_b1, enc_w2, enc_b2, enc_w3, enc_b3, codebook,
           dec_w1, dec_b1, dec_w2, dec_b2, dec_w3, dec_b3):
    f32 = jnp.float32
    imgr = img.reshape(BT, 4)
    imgr4 = img.reshape(B, T, 4)

    w1 = enc_w1[:, 0, :]
    w14 = jnp.zeros((4, 128), f32).at[0:2, 0:H].set(w1.T).at[2:4, H:].set(w1.T)
    b14 = jnp.concatenate([enc_b1, enc_b1])[None]
    w2c = jnp.concatenate([enc_w2[:, :, 0].T, enc_w2[:, :, 1].T], axis=0)
    b2 = enc_b2[None]
    w3 = enc_w3[:, :, 0].T
    b3 = enc_b3[None]
    cbt = codebook.T

    vd0 = dec_w1[:, :, 1]
    vd1 = dec_w1[:, :, 0]
    bd1 = dec_b1[None]
    w2cat = jnp.concatenate([dec_w2[:, :, 1], dec_w2[:, :, 0]], axis=1)
    bd2 = jnp.concatenate([dec_b2, dec_b2])[None]
    wd3 = dec_w3[0, :, 0]
    wsel = jnp.zeros((128, 2), f32).at[0:H, 0].set(wd3).at[H:, 1].set(wd3)
    bd3 = dec_b3.reshape(1, 1)

    idx_flat = _vq_indices(imgr, w14, b14, w2c, b2, w3, b3, cbt)
    q = _sc_gather(codebook, idx_flat.reshape(NW, NCHUNK, CH))
    outr, quant, recon, commit = _decode(
        imgr4, q, w14, b14, w2c, b2, w3, b3, vd0, vd1, bd1, w2cat, bd2, wsel,
        bd3)

    return (outr.reshape(B, 1, L), recon[0, 0], commit[0, 0],
            idx_flat.reshape(B, T), quant)


# speedup vs baseline: 1.2181x; 1.2181x over previous
import jax
import jax.numpy as jnp
from jax import lax
from jax.experimental import pallas as pl
from jax.experimental.pallas import tpu as pltpu
from jax.experimental.pallas import tpu_sc as plsc

B, L, T = 64, 1632, 408
H, D, K = 64, 512, 1024
BT = B * T
TOK = 768
NTILE = BT // TOK
NB = 4
ND = B // NB
TOKC = NB * T
NC, NS = 2, 16
NW = NC * NS
BPW = BT // NW
CH = 48
NCHUNK = BPW // CH
NBUF = 4

_HI = lax.Precision.HIGHEST
_BF = jnp.bfloat16


def _dot(a, b, precision=_HI):
    return jnp.dot(a, b, preferred_element_type=jnp.float32, precision=precision)


def _bdot(a, b):
    return jnp.dot(a.astype(_BF), b.astype(_BF),
                   preferred_element_type=jnp.float32)


def _encode(img_blk, w14, b14, w2c, b2, w3, b3):
    h1 = jnp.maximum(_bdot(img_blk, w14) + b14, 0.0)
    h2 = jnp.maximum(_bdot(h1, w2c) + b2, 0.0)
    return _bdot(h2, w3) + b3


def _enc_vq_body(img_ref, w14_ref, b14_ref, w2_ref, b2_ref, w3_ref, b3_ref,
                 cbt_ref, idx_ref, cbt_bf, c2_ref):
    @pl.when(pl.program_id(0) == 0)
    def _init():
        cbt = cbt_ref[...]
        cbt_bf[...] = (-2.0 * cbt).astype(_BF)
        c2_ref[...] = jnp.sum(cbt * cbt, axis=0, keepdims=True)

    x = _encode(img_ref[...], w14_ref[...], b14_ref[...], w2_ref[...],
                b2_ref[...], w3_ref[...], b3_ref[...])
    e = jnp.dot(x.astype(_BF), cbt_bf[...],
                preferred_element_type=jnp.float32)
    dist = e + c2_ref[...]
    m = jnp.min(dist, axis=1, keepdims=True)
    iota = lax.broadcasted_iota(jnp.int32, dist.shape, 1)
    idx_ref[...] = jnp.min(jnp.where(dist == m, iota, K), axis=1)[None, None]


def _vq_indices(imgr, w14, b14, w2c, b2, w3, b3, cbt):
    rep = lambda i: (0, 0)
    return pl.pallas_call(
        _enc_vq_body,
        grid=(NTILE,),
        in_specs=[
            pl.BlockSpec((TOK, 4), lambda i: (i, 0)),
            pl.BlockSpec((4, 128), rep),
            pl.BlockSpec((1, 128), rep),
            pl.BlockSpec((128, H), rep),
            pl.BlockSpec((1, H), rep),
            pl.BlockSpec((H, D), rep),
            pl.BlockSpec((1, D), rep),
            pl.BlockSpec((D, K), rep),
        ],
        out_specs=pl.BlockSpec((1, 1, TOK), lambda i: (i, 0, 0)),
        out_shape=jax.ShapeDtypeStruct((NTILE, 1, TOK), jnp.int32),
        scratch_shapes=[
            pltpu.VMEM((D, K), _BF),
            pltpu.VMEM((1, K), jnp.float32),
        ],
    )(imgr, w14, b14, w2c, b2, w3, b3, cbt)


def _sc_gather_body(cb_hbm, idx_hbm, out_hbm, idx_v, bufs, sems):
    wid = lax.axis_index("s") * NC + lax.axis_index("c")
    base = wid * BPW
    pltpu.sync_copy(idx_hbm.at[wid], idx_v)

    def issue(j):
        return pltpu.async_copy(cb_hbm.at[idx_v.at[j]], bufs[j % NBUF],
                                sems[j % NBUF])

    handles = {j: issue(j) for j in range(min(NBUF, NCHUNK))}
    for j in range(NCHUNK):
        handles[j].wait()
        pltpu.sync_copy(bufs[j % NBUF], out_hbm.at[pl.ds(base + j * CH, CH)])
        nj = j + NBUF
        if nj < NCHUNK:
            handles[nj] = issue(nj)


def _sc_gather(codebook, idx3):
    def body(cb_hbm, idx_hbm, out_hbm, idx_v, b0, b1, b2, b3, s0, s1, s2, s3):
        _sc_gather_body(cb_hbm, idx_hbm, out_hbm, idx_v,
                        (b0, b1, b2, b3), (s0, s1, s2, s3))

    return pl.kernel(
        body,
        mesh=plsc.VectorSubcoreMesh(core_axis_name="c", subcore_axis_name="s",
                                    num_cores=NC, num_subcores=NS),
        out_type=jax.ShapeDtypeStruct((BT, D), jnp.float32),
        scratch_types=(
            [pltpu.VMEM((NCHUNK, CH), jnp.int32)]
            + [pltpu.VMEM((CH, D), jnp.float32)] * NBUF
            + [pltpu.SemaphoreType.DMA] * NBUF
        ),
    )(codebook, idx3)


def _dec_body(img_ref, q_ref, w14_ref, b14_ref, w2_ref, b2_ref, w3_ref, b3_ref,
              wd1_ref, bd1_ref, w2big_ref, bd2_ref, wsel_ref, bd3_ref,
              out_ref, quant_ref, recon_ref, commit_ref):
    b = pl.program_id(0)
    img_blk = img_ref[...].reshape(TOKC, 4)
    qb = q_ref[...]
    x = _encode(img_blk, w14_ref[...], b14_ref[...], w2_ref[...],
                b2_ref[...], w3_ref[...], b3_ref[...])
    a01 = jnp.maximum(_bdot(qb, wd1_ref[...]) + bd1_ref[...], 0.0)
    f = jnp.maximum(_dot(a01, w2big_ref[...]) + bd2_ref[...], 0.0)
    out_blk = _dot(f, wsel_ref[...]) + bd3_ref[0, 0]
    out_ref[...] = out_blk.reshape(NB, T, 4)
    qb3 = qb.reshape(NB, T, D)
    for r in range(NB):
        quant_ref[r] = jnp.transpose(qb3[r])

    s_r = jnp.sum((img_blk - out_blk) ** 2)
    s_c = jnp.sum((qb - x) ** 2)
    pr = jnp.where(b == 0, 0.0, recon_ref[0, 0])
    pc = jnp.where(b == 0, 0.0, commit_ref[0, 0])
    last = b == ND - 1
    recon_ref[0, 0] = jnp.where(last, (pr + s_r) / (B * L), pr + s_r)
    commit_ref[0, 0] = jnp.where(last, (pc + s_c) / (BT * D), pc + s_c)


def _decode(imgr4, q, w14, b14, w2c, b2, w3, b3, wd1c, bd1c, w2big, bd2big,
            wsel4, bd3):
    rep = lambda b: (0, 0)
    return pl.pallas_call(
        _dec_body,
        grid=(ND,),
        in_specs=[
            pl.BlockSpec((NB, T, 4), lambda b: (b, 0, 0)),
            pl.BlockSpec((TOKC, D), lambda b: (b, 0)),
            pl.BlockSpec((4, 128), rep),
            pl.BlockSpec((1, 128), rep),
            pl.BlockSpec((128, H), rep),
            pl.BlockSpec((1, H), rep),
            pl.BlockSpec((H, D), rep),
            pl.BlockSpec((1, D), rep),
            pl.BlockSpec((D, 128), rep),
            pl.BlockSpec((1, 128), rep),
            pl.BlockSpec((128, 256), rep),
            pl.BlockSpec((1, 256), rep),
            pl.BlockSpec((256, 4), rep),
            pl.BlockSpec((1, 1), rep),
        ],
        out_specs=[
            pl.BlockSpec((NB, T, 4), lambda b: (b, 0, 0)),
            pl.BlockSpec((NB, D, T), lambda b: (b, 0, 0)),
            pl.BlockSpec(memory_space=pltpu.SMEM),
            pl.BlockSpec(memory_space=pltpu.SMEM),
        ],
        out_shape=[
            jax.ShapeDtypeStruct((B, T, 4), jnp.float32),
            jax.ShapeDtypeStruct((B, D, T), jnp.float32),
            jax.ShapeDtypeStruct((1, 1), jnp.float32),
            jax.ShapeDtypeStruct((1, 1), jnp.float32),
        ],
    )(imgr4, q, w14, b14, w2c, b2, w3, b3, wd1c, bd1c, w2big, bd2big, wsel4,
      bd3)


def kernel(img, enc_w1, enc_b1, enc_w2, enc_b2, enc_w3, enc_b3, codebook,
           dec_w1, dec_b1, dec_w2, dec_b2, dec_w3, dec_b3):
    f32 = jnp.float32
    imgr = img.reshape(BT, 4)
    imgr4 = img.reshape(B, T, 4)

    w1 = enc_w1[:, 0, :]
    w14 = jnp.zeros((4, 128), f32).at[0:2, 0:H].set(w1.T).at[2:4, H:].set(w1.T)
    b14 = jnp.concatenate([enc_b1, enc_b1])[None]
    w2c = jnp.concatenate([enc_w2[:, :, 0].T, enc_w2[:, :, 1].T], axis=0)
    b2 = enc_b2[None]
    w3 = enc_w3[:, :, 0].T
    b3 = enc_b3[None]
    cbt = codebook.T

    wd1c = jnp.concatenate([dec_w1[:, :, 1], dec_w1[:, :, 0]], axis=1)
    bd1c = jnp.concatenate([dec_b1, dec_b1])[None]
    w2cat = jnp.concatenate([dec_w2[:, :, 1], dec_w2[:, :, 0]], axis=1)
    w2big = jnp.zeros((128, 256), f32).at[0:H, 0:128].set(w2cat) \
                                      .at[H:, 128:].set(w2cat)
    bd2big = jnp.concatenate([dec_b2, dec_b2, dec_b2, dec_b2])[None]
    wd3 = dec_w3[0, :, 0]
    wsel4 = jnp.zeros((256, 4), f32)
    for r in range(4):
        wsel4 = wsel4.at[r * H:(r + 1) * H, r].set(wd3)
    bd3 = dec_b3.reshape(1, 1)

    idx_flat = _vq_indices(imgr, w14, b14, w2c, b2, w3, b3, cbt)
    q = _sc_gather(codebook, idx_flat.reshape(NW, NCHUNK, CH))
    outr, quant, recon, commit = _decode(
        imgr4, q, w14, b14, w2c, b2, w3, b3, wd1c, bd1c, w2big, bd2big, wsel4,
        bd3)

    return (outr.reshape(B, 1, L), recon[0, 0], commit[0, 0],
            idx_flat.reshape(B, T), quant)
